# Initial kernel scaffold; baseline (speedup 1.0000x reference)
import jax
import jax.numpy as jnp
from jax.experimental import pallas as pl


def kernel(input, enc_w0, enc_b0, enc_w1, enc_b1, enc_w2, enc_b2, enc_w3, enc_b3, embed, dec_w0, dec_b0, dec_tw1, dec_tb1, dec_tw2, dec_tb2, dec_tw3, dec_tb3):
    raise NotImplementedError("write your pallas kernel here")


# speedup vs baseline: 1.4223x; 1.4223x over previous
import jax
import jax.numpy as jnp
from jax import lax
from jax.experimental import pallas as pl

BF = jnp.bfloat16
F32 = jnp.float32
B = 8
U = 1024
C = 256
NE = 2048


def _lrelu(x):
    return jnp.where(x >= 0, x, 0.2 * x)


def _mm(x, w):
    return jnp.dot(x.astype(BF), w.astype(BF), preferred_element_type=F32)


def _sdn(x):
    return jnp.concatenate([jnp.zeros((1, x.shape[1]), F32), x[:-1]], axis=0)


def _sup(x):
    return jnp.concatenate([x[1:], jnp.zeros((1, x.shape[1]), F32)], axis=0)


def _conv_taps(parts, wt, bias):
    acc = _mm(parts[0], wt[0])
    for t in range(1, len(parts)):
        acc = acc + _mm(parts[t], wt[t])
    return acc + bias[None, :]


def _body(x_ref, w0_ref, b0_ref, wt1_ref, b1_ref, wt2_ref, b2_ref,
          wt3_ref, b3_ref, emb_ref, e2_ref, ehi_ref, elo_ref,
          dw0_ref, db0_ref, dt1_ref, db1_ref, dt2_ref, db2_ref, dt3_ref,
          out_ref, dsum_ref):
    X = x_ref[0]
    xm1 = jnp.concatenate([jnp.zeros((1, 1), F32), X[:-1, 7:8]], axis=0)
    xp1 = jnp.concatenate([X[1:, 0:1], jnp.zeros((1, 1), F32)], axis=0)
    xext = jnp.concatenate([xm1, X, xp1], axis=1)
    y0 = _mm(xext, w0_ref[...]) + b0_ref[...][None, :]
    y0 = _lrelu(y0)
    h1 = [y0[:, q * C:(q + 1) * C] for q in range(4)]

    wt1 = wt1_ref[...]
    h2_0 = _lrelu(_conv_taps([_sdn(h1[3]), h1[0], h1[1], h1[2]],
                             [wt1[0], wt1[1], wt1[2], wt1[3]], b1_ref[...]))
    h2_1 = _lrelu(_conv_taps([h1[1], h1[2], h1[3], _sup(h1[0])],
                             [wt1[0], wt1[1], wt1[2], wt1[3]], b1_ref[...]))

    wt2 = wt2_ref[...]
    h3 = _lrelu(_conv_taps([_sdn(h2_1), h2_0, h2_1, _sup(h2_0)],
                           [wt2[0], wt2[1], wt2[2], wt2[3]], b2_ref[...]))

    wt3 = wt3_ref[...]
    z = _conv_taps([_sdn(h3), h3, _sup(h3)],
                   [wt3[0], wt3[1], wt3[2]], b3_ref[...])

    m = _mm(z, emb_ref[...])
    z2 = jnp.sum(z * z, axis=1, keepdims=True)
    dist = (z2 - 2.0 * m) + e2_ref[...][None, :]
    dmin = jnp.min(dist, axis=1, keepdims=True)
    ids = lax.broadcasted_iota(jnp.int32, (U, NE), 1)
    idx = jnp.min(jnp.where(dist == dmin, ids, NE), axis=1, keepdims=True)
    onehot = (ids == idx).astype(BF)
    q = (jnp.dot(onehot, ehi_ref[...], preferred_element_type=F32)
         + jnp.dot(onehot, elo_ref[...], preferred_element_type=F32))

    dd = q - z
    dsum_ref[0, 0] = jnp.sum(dd * dd) + jnp.zeros((128,), F32)

    dw0 = dw0_ref[...]
    d0 = _lrelu(_conv_taps([_sdn(q), q, _sup(q)],
                           [dw0[0], dw0[1], dw0[2]], db0_ref[...]))

    dt1 = dt1_ref[...]
    d1_0 = _lrelu((_mm(d0, dt1[1]) + _mm(_sdn(d0), dt1[3])) + db1_ref[...][None, :])
    d1_1 = _lrelu((_mm(_sup(d0), dt1[0]) + _mm(d0, dt1[2])) + db1_ref[...][None, :])

    dt2 = dt2_ref[...]
    bb = db2_ref[...][None, :]
    d2_0 = _lrelu((_mm(d1_0, dt2[1]) + _mm(_sdn(d1_1), dt2[3])) + bb)
    d2_1 = _lrelu((_mm(d1_1, dt2[0]) + _mm(d1_0, dt2[2])) + bb)
    d2_2 = _lrelu((_mm(d1_1, dt2[1]) + _mm(d1_0, dt2[3])) + bb)
    d2_3 = _lrelu((_mm(_sup(d1_0), dt2[0]) + _mm(d1_1, dt2[2])) + bb)
    d2 = [d2_0, d2_1, d2_2, d2_3]

    A = [_mm(v, dt3_ref[...]) for v in d2]
    A3dn = jnp.concatenate([jnp.zeros((1, 4), F32), A[3][:-1]], axis=0)
    A0up = jnp.concatenate([A[0][1:], jnp.zeros((1, 4), F32)], axis=0)
    cols = []
    for p in range(8):
        if p % 2 == 0:
            v = p // 2
            prev = A3dn if v == 0 else A[v - 1]
            cols.append(A[v][:, 1:2] + prev[:, 3:4])
        else:
            v = (p - 1) // 2
            nxt = A0up if v == 3 else A[v + 1]
            cols.append(nxt[:, 0:1] + A[v][:, 2:3])
    out_ref[0] = jnp.concatenate(cols, axis=1)


def kernel(input, enc_w0, enc_b0, enc_w1, enc_b1, enc_w2, enc_b2, enc_w3, enc_b3,
           embed, dec_w0, dec_b0, dec_tw1, dec_tb1, dec_tw2, dec_tb2, dec_tw3, dec_tb3):
    X = input.reshape(B, U, 8)

    w0t = enc_w0[:, 0, :]
    rows = []
    for p in range(-1, 9):
        colw = []
        for qq in range(4):
            j = p - (2 * qq - 1)
            colw.append(w0t[:, j] if 0 <= j < 4 else jnp.zeros((C,), F32))
        rows.append(jnp.concatenate(colw, 0))
    w0mat = jnp.stack(rows, 0)
    b0row = jnp.tile(enc_b0, 4)

    wt1 = jnp.stack([enc_w1[:, :, j].T for j in range(4)], 0)
    wt2 = jnp.stack([enc_w2[:, :, j].T for j in range(4)], 0)
    wt3 = jnp.stack([enc_w3[:, :, j].T for j in range(3)], 0)

    e2 = (embed ** 2).sum(0)
    embT = embed.T
    ehi = embT.astype(BF)
    elo = (embT - ehi.astype(F32)).astype(BF)

    dw0 = jnp.stack([dec_w0[:, :, j].T for j in range(3)], 0)
    dt1 = jnp.stack([dec_tw1[:, :, k] for k in range(4)], 0)
    dt2 = jnp.stack([dec_tw2[:, :, k] for k in range(4)], 0)
    dt3 = dec_tw3[:, 0, :]

    def full(s):
        return pl.BlockSpec(s, lambda i: tuple(0 for _ in s))

    out, dsum = pl.pallas_call(
        _body,
        grid=(B,),
        in_specs=[pl.BlockSpec((1, U, 8), lambda i: (i, 0, 0)),
                  full((10, 4 * C)), full((4 * C,)),
                  full((4, C, C)), full((C,)),
                  full((4, C, C)), full((C,)),
                  full((3, C, C)), full((C,)),
                  full((C, NE)), full((NE,)),
                  full((NE, C)), full((NE, C)),
                  full((3, C, C)), full((C,)),
                  full((4, C, C)), full((C,)),
                  full((4, C, C)), full((C,)),
                  full((C, 4))],
        out_specs=[pl.BlockSpec((1, U, 8), lambda i: (i, 0, 0)),
                   pl.BlockSpec((1, 1, 128), lambda i: (i, 0, 0))],
        out_shape=[jax.ShapeDtypeStruct((B, U, 8), F32),
                   jax.ShapeDtypeStruct((B, 1, 128), F32)],
    )(X, w0mat, b0row, wt1, enc_b1, wt2, enc_b2, wt3, enc_b3,
      embed, e2, ehi, elo, dw0, dec_b0, dt1, dec_tb1, dt2, dec_tb2, dt3)

    dec = (out + dec_tb3[0]).reshape(B, U * 8, 1)
    diff = jnp.sum(dsum[:, 0, 0]) / (B * U * C)
    return dec, diff
